# baseline (speedup 1.0000x reference)
import functools

import jax
import jax.numpy as jnp
from jax import lax
from jax.experimental import pallas as pl
from jax.experimental.pallas import tpu as pltpu
from jax.experimental.pallas import tpu_sc as plsc

N = 10000
E = 160000
D = 256
G = 64

NC = 2
NS = 16
LANES = 16
NW = NC * NS
CHP = 50
NCHP = 100
EPT = E // NW
NP = N
RPA = 624
REM = N - NS * RPA
ZCH = 48
EPS = 1e-5

_HIGH = lax.Precision.HIGHEST


def _mesh():
    return plsc.VectorSubcoreMesh(core_axis_name="c", subcore_axis_name="s",
                                  num_cores=NC, num_subcores=NS)


@functools.lru_cache(maxsize=None)
def _make_deg():
    @functools.partial(
        pl.kernel,
        out_type=jax.ShapeDtypeStruct((NC, N, LANES), jnp.float32),
        mesh=_mesh(),
        scratch_types=[
            pltpu.VMEM((NCHP, CHP), jnp.int32),
            pltpu.VMEM((CHP, LANES), jnp.float32),
            pltpu.VMEM_SHARED((NP, LANES), jnp.float32),
            pltpu.SemaphoreType.DMA,
        ],
        compiler_params=pltpu.CompilerParams(use_tc_tiling_on_sc=False),
    )
    def deg_kernel(ei_hbm, out_hbm, dstb, ones_v, table, sem):
        c = lax.axis_index("c")
        s = lax.axis_index("s")
        wid = s * NC + c
        pltpu.sync_copy(ei_hbm.at[1, pl.ds(wid * NCHP, NCHP)], dstb)
        zeros16 = jnp.zeros((LANES,), jnp.float32)

        def zrow(r, carry):
            ones_v[r, :] = zeros16
            return carry

        lax.fori_loop(0, CHP, zrow, None)
        for k in range(RPA // ZCH):
            pltpu.async_copy(ones_v.at[pl.ds(0, ZCH)],
                             table.at[pl.ds(s * RPA + k * ZCH, ZCH)], sem)
        for k in range(RPA // ZCH):
            pltpu.make_async_copy(
                ones_v.at[pl.ds(0, ZCH)],
                table.at[pl.ds(s * RPA + k * ZCH, ZCH)], sem).wait()

        @pl.when(s == 0)
        def _ztail():
            pltpu.sync_copy(ones_v.at[pl.ds(0, REM)],
                            table.at[pl.ds(NS * RPA, REM)])

        ones16 = jnp.ones((LANES,), jnp.float32)

        def orow(r, carry):
            ones_v[r, :] = ones16
            return carry

        lax.fori_loop(0, CHP, orow, None)
        plsc.subcore_barrier()

        def group(g, carry):
            def fire(j, carry2):
                pltpu.async_copy(ones_v, table.at[dstb.at[g * 10 + j]], sem,
                                 add=True)
                return carry2

            lax.fori_loop(0, 10, fire, None)

            def drain(j, carry2):
                pltpu.make_async_copy(ones_v, table.at[dstb.at[g * 10 + j]],
                                      sem).wait()
                return carry2

            lax.fori_loop(0, 10, drain, None)
            return carry

        lax.fori_loop(0, NCHP // 10, group, None)

        plsc.subcore_barrier()
        pltpu.sync_copy(table.at[pl.ds(s * RPA, RPA)],
                        out_hbm.at[c, pl.ds(s * RPA, RPA)])

        @pl.when(s == 0)
        def _rtail():
            pltpu.sync_copy(table.at[pl.ds(NS * RPA, REM)],
                            out_hbm.at[c, pl.ds(NS * RPA, REM)])

    return deg_kernel


@functools.lru_cache(maxsize=None)
def _make_prop(F):
    @functools.partial(
        pl.kernel,
        out_type=jax.ShapeDtypeStruct((NC, N, F), jnp.float32),
        mesh=_mesh(),
        scratch_types=[
            pltpu.VMEM((NCHP, CHP), jnp.int32),
            pltpu.VMEM((NCHP, CHP), jnp.int32),
            pltpu.VMEM((CHP, F), jnp.float32),
            pltpu.VMEM((CHP, F), jnp.float32),
            pltpu.VMEM((CHP, F), jnp.float32),
            pltpu.VMEM((CHP, F), jnp.float32),
            pltpu.SemaphoreType.DMA,
            pltpu.SemaphoreType.DMA,
            pltpu.SemaphoreType.DMA,
            pltpu.SemaphoreType.DMA,
            pltpu.SemaphoreType.DMA,
            pltpu.SemaphoreType.DMA,
            pltpu.SemaphoreType.DMA,
            pltpu.SemaphoreType.DMA,
            pltpu.VMEM_SHARED((NP, F), jnp.float32),
            pltpu.SemaphoreType.DMA,
        ],
        compiler_params=pltpu.CompilerParams(use_tc_tiling_on_sc=False),
    )
    def prop_kernel(xz_hbm, ei_hbm, out_hbm,
                    srcb, dstb, r0, r1, r2, r3, g0, g1, g2, g3,
                    s0, s1, s2, s3, accum, sem):
        rows = (r0, r1, r2, r3)
        gsem = (g0, g1, g2, g3)
        ssem = (s0, s1, s2, s3)
        c = lax.axis_index("c")
        s = lax.axis_index("s")
        wid = s * NC + c
        pltpu.async_copy(ei_hbm.at[0, pl.ds(wid * NCHP, NCHP)], srcb, sem)
        pltpu.sync_copy(ei_hbm.at[1, pl.ds(wid * NCHP, NCHP)], dstb)
        pltpu.make_async_copy(ei_hbm.at[0, pl.ds(wid * NCHP, NCHP)], srcb,
                              sem).wait()
        zeros16 = jnp.zeros((LANES,), jnp.float32)

        def zrow(r, carry):
            for j in range(F // LANES):
                rows[0][r, pl.ds(j * LANES, LANES)] = zeros16
            return carry

        lax.fori_loop(0, ZCH, zrow, None)
        for k in range(RPA // ZCH):
            pltpu.async_copy(rows[0].at[pl.ds(0, ZCH)],
                             accum.at[pl.ds(s * RPA + k * ZCH, ZCH)], sem)
        for k in range(RPA // ZCH):
            pltpu.make_async_copy(
                rows[0].at[pl.ds(0, ZCH)],
                accum.at[pl.ds(s * RPA + k * ZCH, ZCH)], sem).wait()

        @pl.when(s == 0)
        def _ztail():
            pltpu.sync_copy(rows[0].at[pl.ds(0, REM)],
                            accum.at[pl.ds(NS * RPA, REM)])

        plsc.subcore_barrier()

        pltpu.async_copy(xz_hbm.at[srcb.at[0]], rows[0], gsem[0])
        pltpu.async_copy(xz_hbm.at[srcb.at[1]], rows[1], gsem[1])

        def group(g, carry):
            for b in range(4):
                i = 4 * g + b
                b2 = (b + 2) % 4

                @pl.when(i >= 2)
                def _drain():
                    pltpu.make_async_copy(
                        rows[b2], accum.at[dstb.at[i - 2]], ssem[b2]).wait()

                @pl.when(i + 2 < NCHP)
                def _prefetch():
                    pltpu.async_copy(xz_hbm.at[srcb.at[i + 2]], rows[b2],
                                     gsem[b2])

                pltpu.make_async_copy(xz_hbm.at[srcb.at[i]], rows[b],
                                      gsem[b]).wait()
                pltpu.async_copy(rows[b], accum.at[dstb.at[i]], ssem[b],
                                 add=True)
            return carry

        lax.fori_loop(0, NCHP // 4, group, None)
        for b in (2, 3):
            pltpu.make_async_copy(rows[b], accum.at[dstb.at[NCHP - 4 + b]],
                                  ssem[b]).wait()

        plsc.subcore_barrier()
        pltpu.sync_copy(accum.at[pl.ds(s * RPA, RPA)],
                        out_hbm.at[c, pl.ds(s * RPA, RPA)])

        @pl.when(s == 0)
        def _rtail():
            pltpu.sync_copy(accum.at[pl.ds(NS * RPA, REM)],
                            out_hbm.at[c, pl.ds(NS * RPA, REM)])

    return prop_kernel


_BR = 1000
_GRID = N // _BR


def _mm1_body(x_ref, degt_ref, w1_ref, xz_ref, dis_ref):
    dsum = (jnp.sum(degt_ref[0], axis=1, keepdims=True)
            + jnp.sum(degt_ref[1], axis=1, keepdims=True))
    deg = 1.0 + dsum / float(LANES)
    dis = lax.rsqrt(deg)
    xw = jnp.dot(x_ref[...], w1_ref[...],
                 preferred_element_type=jnp.float32, precision=_HIGH)
    xz_ref[...] = xw * dis
    dis_ref[...] = dis


def _lm1_body(part_ref, xz_ref, dis_ref, b_ref, g_ref, be_ref, w2_ref,
              xz2_ref, a1_s, st_s):
    i = pl.program_id(0)

    @pl.when(i < _GRID)
    def _phase_a():
        acc = part_ref[0] + part_ref[1] + xz_ref[...]
        pre = acc * dis_ref[...] + b_ref[...][None, :]
        a = jnp.maximum(pre, 0.0)
        a1_s[pl.ds(i * _BR, _BR), :] = a
        st = jnp.stack([jnp.sum(a, axis=0), jnp.sum(a * a, axis=0)], axis=0)

        @pl.when(i == 0)
        def _init():
            st_s[...] = st

        @pl.when(i != 0)
        def _acc():
            st_s[...] = st_s[...] + st

    @pl.when(i >= _GRID)
    def _phase_b():
        j = i - _GRID
        mu = st_s[0] / float(N)
        var = st_s[1] / float(N) - mu * mu
        scale = g_ref[...] * lax.rsqrt(var + EPS)
        shift = be_ref[...] - mu * scale
        h = a1_s[pl.ds(j * _BR, _BR), :] * scale[None, :] + shift[None, :]
        xw2 = jnp.dot(h, w2_ref[...],
                      preferred_element_type=jnp.float32, precision=_HIGH)
        xz2_ref[...] = xw2 * dis_ref[...]


def _lh_body(part_ref, xz_ref, dis_ref, b_ref, batch_ref,
             g2_ref, be2_ref, wf1_ref, bf1_ref, g3_ref, be3_ref,
             wf2_ref, bf2_ref, wf3_ref, bf3_ref,
             out_ref, st_s, q_s, cnt_s):
    i = pl.program_id(0)

    @pl.when(i < _GRID)
    def _phase_pool():
        acc = part_ref[0] + part_ref[1] + xz_ref[...]
        pre = acc * dis_ref[...] + b_ref[...][None, :]
        a = jnp.maximum(pre, 0.0)
        st = jnp.stack([jnp.sum(a, axis=0), jnp.sum(a * a, axis=0)], axis=0)
        gids = lax.broadcasted_iota(jnp.int32, (_BR, G), 1)
        oh = (batch_ref[...] == gids).astype(jnp.float32)
        q = lax.dot_general(oh, a, (((0,), (0,)), ((), ())),
                            preferred_element_type=jnp.float32,
                            precision=_HIGH)
        cnt = jnp.sum(oh, axis=0)[None, :]

        @pl.when(i == 0)
        def _init():
            st_s[...] = st
            q_s[...] = q
            cnt_s[...] = cnt

        @pl.when(i != 0)
        def _acc():
            st_s[...] = st_s[...] + st
            q_s[...] = q_s[...] + q
            cnt_s[...] = cnt_s[...] + cnt

    @pl.when(i == _GRID)
    def _phase_head():
        mu = st_s[0] / float(N)
        var = st_s[1] / float(N) - mu * mu
        scale = g2_ref[...] * lax.rsqrt(var + EPS)
        shift = be2_ref[...] - mu * scale
        p = q_s[...] * scale[None, :] + cnt_s[0][:, None] * shift[None, :]
        p = jnp.maximum(jnp.dot(p, wf1_ref[...],
                                preferred_element_type=jnp.float32,
                                precision=_HIGH) + bf1_ref[...][None, :], 0.0)
        mu3 = jnp.mean(p, axis=0)
        var3 = jnp.mean((p - mu3[None, :]) ** 2, axis=0)
        p = g3_ref[...][None, :] * (p - mu3[None, :]) * lax.rsqrt(var3 + EPS) \
            + be3_ref[...][None, :]
        p = jnp.maximum(jnp.dot(p, wf2_ref[...],
                                preferred_element_type=jnp.float32,
                                precision=_HIGH) + bf2_ref[...][None, :], 0.0)
        z = jnp.dot(p, wf3_ref[...],
                    preferred_element_type=jnp.float32,
                    precision=_HIGH) + bf3_ref[...][None, :]
        m = jnp.max(z, axis=1, keepdims=True)
        out_ref[...] = z - m - jnp.log(jnp.sum(jnp.exp(z - m), axis=1,
                                               keepdims=True))


def _full(shape):
    return pl.BlockSpec(shape, lambda i: tuple(0 for _ in shape))


def _rows(bshape):
    return pl.BlockSpec(bshape, lambda i: (i,) + tuple(0 for _ in bshape[1:]))


def _tc_mm1(x, degt, w1):
    return pl.pallas_call(
        _mm1_body,
        grid=(_GRID,),
        in_specs=[
            _rows((_BR, D)),
            pl.BlockSpec((NC, _BR, LANES), lambda i: (0, i, 0)),
            _full((D, 128)),
        ],
        out_specs=[_rows((_BR, 128)), _rows((_BR, 1))],
        out_shape=[jax.ShapeDtypeStruct((N, 128), jnp.float32),
                   jax.ShapeDtypeStruct((N, 1), jnp.float32)],
    )(x, degt, w1)


def _frz(bshape):
    return pl.BlockSpec(
        bshape,
        lambda i: (jnp.where(i < _GRID, i, _GRID - 1),)
        + tuple(0 for _ in bshape[1:]))


def _frz3(bshape):
    return pl.BlockSpec(
        bshape,
        lambda i: (0, jnp.where(i < _GRID, i, _GRID - 1))
        + tuple(0 for _ in bshape[2:]))


def _tc_lm1(part, xz, dis, b1, g1, be1, w2):
    return pl.pallas_call(
        _lm1_body,
        grid=(2 * _GRID,),
        in_specs=[
            _frz3((NC, _BR, 128)),
            _frz((_BR, 128)),
            pl.BlockSpec((_BR, 1), lambda i: (i % _GRID, 0)),
            _full((128,)),
            _full((128,)),
            _full((128,)),
            _full((128, 64)),
        ],
        out_specs=pl.BlockSpec((_BR, 64), lambda i: (i % _GRID, 0)),
        out_shape=jax.ShapeDtypeStruct((N, 64), jnp.float32),
        scratch_shapes=[pltpu.VMEM((N, 128), jnp.float32),
                        pltpu.VMEM((2, 128), jnp.float32)],
    )(part, xz, dis, b1, g1, be1, w2)


def _tc_lh(part, xz2, dis, b2, batch,
           g2, be2, wf1, bf1, g3, be3, wf2, bf2, wf3, bf3):
    return pl.pallas_call(
        _lh_body,
        grid=(_GRID + 1,),
        in_specs=[
            _frz3((NC, _BR, 64)),
            _frz((_BR, 64)),
            _frz((_BR, 1)),
            _full((64,)),
            _frz((_BR, 1)),
            _full((64,)), _full((64,)),
            _full((64, 64)), _full((64,)), _full((64,)), _full((64,)),
            _full((64, 64)), _full((64,)), _full((64, 2)), _full((2,)),
        ],
        out_specs=pl.BlockSpec((G, 2), lambda i: (0, 0)),
        out_shape=jax.ShapeDtypeStruct((G, 2), jnp.float32),
        scratch_shapes=[pltpu.VMEM((2, 64), jnp.float32),
                        pltpu.VMEM((G, 64), jnp.float32),
                        pltpu.VMEM((1, 64), jnp.float32)],
    )(part, xz2, dis, b2, batch,
      g2, be2, wf1, bf1, g3, be3, wf2, bf2, wf3, bf3)


def kernel(x, edge_index, batch, W1, b1, g1, be1, W2, b2, g2, be2,
           Wf1, bf1, g3, be3, Wf2, bf2, Wf3, bf3):
    ei3 = edge_index.reshape(2, NW * NCHP, CHP)
    degt = _make_deg()(ei3)
    xz1, dis = _tc_mm1(x, degt, W1)
    part1 = _make_prop(128)(xz1, ei3)
    xz2 = _tc_lm1(part1, xz1, dis, b1, g1, be1, W2)
    part2 = _make_prop(64)(xz2, ei3)
    return _tc_lh(part2, xz2, dis, b2, batch[:, None],
                  g2, be2, Wf1, bf1, g3, be3, Wf2, bf2, Wf3, bf3)


# speedup vs baseline: 1.0236x; 1.0236x over previous
import functools

import jax
import jax.numpy as jnp
from jax import lax
from jax.experimental import pallas as pl
from jax.experimental.pallas import tpu as pltpu
from jax.experimental.pallas import tpu_sc as plsc

N = 10000
E = 160000
D = 256
G = 64

NC = 2
NS = 16
LANES = 16
NW = NC * NS
CH = 128
NCH = 40
CHP = 64
NCHP = 80
EPT = E // NW
PADPT = NCH * CH - EPT
DUMP = 4
NP = N + NS * DUMP
RPA = 624
REM = N - NS * RPA
ZCH = 48
EPS = 1e-5

_HIGH = lax.Precision.HIGHEST


def _mesh():
    return plsc.VectorSubcoreMesh(core_axis_name="c", subcore_axis_name="s",
                                  num_cores=NC, num_subcores=NS)


@functools.lru_cache(maxsize=None)
def _make_deg():
    @functools.partial(
        pl.kernel,
        out_type=jax.ShapeDtypeStruct((NC, N, LANES), jnp.float32),
        mesh=_mesh(),
        scratch_types=[
            pltpu.VMEM((NCH, CH), jnp.int32),
            pltpu.VMEM((CH, LANES), jnp.float32),
            pltpu.VMEM_SHARED((NP, LANES), jnp.float32),
            pltpu.SemaphoreType.DMA,
        ],
        compiler_params=pltpu.CompilerParams(use_tc_tiling_on_sc=False),
    )
    def deg_kernel(dst_hbm, out_hbm, dstb, ones_v, table, sem):
        c = lax.axis_index("c")
        s = lax.axis_index("s")
        wid = s * NC + c
        pltpu.sync_copy(dst_hbm.at[pl.ds(wid * NCH, NCH)], dstb)
        zeros16 = jnp.zeros((LANES,), jnp.float32)

        def zrow(r, carry):
            ones_v[r, :] = zeros16
            return carry

        lax.fori_loop(0, CH, zrow, None)
        for k in range(RPA // ZCH):
            pltpu.async_copy(ones_v.at[pl.ds(0, ZCH)],
                             table.at[pl.ds(s * RPA + k * ZCH, ZCH)], sem)
        for k in range(RPA // ZCH):
            pltpu.make_async_copy(
                ones_v.at[pl.ds(0, ZCH)],
                table.at[pl.ds(s * RPA + k * ZCH, ZCH)], sem).wait()

        @pl.when(s == 0)
        def _ztail():
            pltpu.sync_copy(ones_v.at[pl.ds(0, REM)],
                            table.at[pl.ds(NS * RPA, REM)])

        ones16 = jnp.ones((LANES,), jnp.float32)

        def orow(r, carry):
            ones_v[r, :] = ones16
            return carry

        lax.fori_loop(0, CH, orow, None)
        plsc.subcore_barrier()

        def group(g, carry):
            def fire(j, carry2):
                pltpu.async_copy(ones_v, table.at[dstb.at[g * 8 + j]], sem,
                                 add=True)
                return carry2

            lax.fori_loop(0, 8, fire, None)

            def drain(j, carry2):
                pltpu.make_async_copy(ones_v, table.at[dstb.at[g * 8 + j]],
                                      sem).wait()
                return carry2

            lax.fori_loop(0, 8, drain, None)
            return carry

        lax.fori_loop(0, NCH // 8, group, None)

        plsc.subcore_barrier()
        pltpu.sync_copy(table.at[pl.ds(s * RPA, RPA)],
                        out_hbm.at[c, pl.ds(s * RPA, RPA)])

        @pl.when(s == 0)
        def _rtail():
            pltpu.sync_copy(table.at[pl.ds(NS * RPA, REM)],
                            out_hbm.at[c, pl.ds(NS * RPA, REM)])

    return deg_kernel


@functools.lru_cache(maxsize=None)
def _make_prop(F):
    @functools.partial(
        pl.kernel,
        out_type=jax.ShapeDtypeStruct((NC, N, F), jnp.float32),
        mesh=_mesh(),
        scratch_types=[
            pltpu.VMEM((NCHP, CHP), jnp.int32),
            pltpu.VMEM((NCHP, CHP), jnp.int32),
            pltpu.VMEM((CHP, F), jnp.float32),
            pltpu.VMEM((CHP, F), jnp.float32),
            pltpu.VMEM((CHP, F), jnp.float32),
            pltpu.VMEM((CHP, F), jnp.float32),
            pltpu.SemaphoreType.DMA,
            pltpu.SemaphoreType.DMA,
            pltpu.SemaphoreType.DMA,
            pltpu.SemaphoreType.DMA,
            pltpu.SemaphoreType.DMA,
            pltpu.SemaphoreType.DMA,
            pltpu.SemaphoreType.DMA,
            pltpu.SemaphoreType.DMA,
            pltpu.VMEM_SHARED((NP, F), jnp.float32),
            pltpu.SemaphoreType.DMA,
        ],
        compiler_params=pltpu.CompilerParams(use_tc_tiling_on_sc=False),
    )
    def prop_kernel(xz_hbm, src_hbm, dst_hbm, out_hbm,
                    srcb, dstb, r0, r1, r2, r3, g0, g1, g2, g3,
                    s0, s1, s2, s3, accum, sem):
        rows = (r0, r1, r2, r3)
        gsem = (g0, g1, g2, g3)
        ssem = (s0, s1, s2, s3)
        c = lax.axis_index("c")
        s = lax.axis_index("s")
        wid = s * NC + c
        pltpu.async_copy(src_hbm.at[pl.ds(wid * NCHP, NCHP)], srcb, sem)
        pltpu.sync_copy(dst_hbm.at[pl.ds(wid * NCHP, NCHP)], dstb)
        pltpu.make_async_copy(src_hbm.at[pl.ds(wid * NCHP, NCHP)], srcb,
                              sem).wait()
        zeros16 = jnp.zeros((LANES,), jnp.float32)

        def zrow(r, carry):
            for j in range(F // LANES):
                rows[0][r, pl.ds(j * LANES, LANES)] = zeros16
            return carry

        lax.fori_loop(0, ZCH, zrow, None)
        for k in range(RPA // ZCH):
            pltpu.async_copy(rows[0].at[pl.ds(0, ZCH)],
                             accum.at[pl.ds(s * RPA + k * ZCH, ZCH)], sem)
        for k in range(RPA // ZCH):
            pltpu.make_async_copy(
                rows[0].at[pl.ds(0, ZCH)],
                accum.at[pl.ds(s * RPA + k * ZCH, ZCH)], sem).wait()

        @pl.when(s == 0)
        def _ztail():
            pltpu.sync_copy(rows[0].at[pl.ds(0, REM + 8)],
                            accum.at[pl.ds(NS * RPA, REM + 8)])

        plsc.subcore_barrier()

        pltpu.async_copy(xz_hbm.at[srcb.at[0]], rows[0], gsem[0])
        pltpu.async_copy(xz_hbm.at[srcb.at[1]], rows[1], gsem[1])

        def group(g, carry):
            for b in range(4):
                i = 4 * g + b
                b2 = (b + 2) % 4

                @pl.when(i >= 2)
                def _drain():
                    pltpu.make_async_copy(
                        rows[b2], accum.at[dstb.at[i - 2]], ssem[b2]).wait()

                @pl.when(i + 2 < NCHP)
                def _prefetch():
                    pltpu.async_copy(xz_hbm.at[srcb.at[i + 2]], rows[b2],
                                     gsem[b2])

                pltpu.make_async_copy(xz_hbm.at[srcb.at[i]], rows[b],
                                      gsem[b]).wait()
                pltpu.async_copy(rows[b], accum.at[dstb.at[i]], ssem[b],
                                 add=True)
            return carry

        lax.fori_loop(0, NCHP // 4, group, None)
        for b in (2, 3):
            pltpu.make_async_copy(rows[b], accum.at[dstb.at[NCHP - 4 + b]],
                                  ssem[b]).wait()

        plsc.subcore_barrier()
        pltpu.sync_copy(accum.at[pl.ds(s * RPA, RPA)],
                        out_hbm.at[c, pl.ds(s * RPA, RPA)])

        @pl.when(s == 0)
        def _rtail():
            pltpu.sync_copy(accum.at[pl.ds(NS * RPA, REM)],
                            out_hbm.at[c, pl.ds(NS * RPA, REM)])

    return prop_kernel


_BR = 1000
_GRID = N // _BR


def _mm1_body(x_ref, degt_ref, w1_ref, xz_ref, dis_ref):
    dsum = (jnp.sum(degt_ref[0], axis=1, keepdims=True)
            + jnp.sum(degt_ref[1], axis=1, keepdims=True))
    deg = 1.0 + dsum / float(LANES)
    dis = lax.rsqrt(deg)
    xw = jnp.dot(x_ref[...], w1_ref[...],
                 preferred_element_type=jnp.float32, precision=_HIGH)
    xz_ref[...] = xw * dis
    dis_ref[...] = dis


def _lm1_body(part_ref, xz_ref, dis_ref, b_ref, g_ref, be_ref, w2_ref,
              xz2_ref, a1_s, st_s):
    i = pl.program_id(0)

    @pl.when(i < _GRID)
    def _phase_a():
        acc = part_ref[0] + part_ref[1] + xz_ref[...]
        pre = acc * dis_ref[...] + b_ref[...][None, :]
        a = jnp.maximum(pre, 0.0)
        a1_s[pl.ds(i * _BR, _BR), :] = a
        st = jnp.stack([jnp.sum(a, axis=0), jnp.sum(a * a, axis=0)], axis=0)

        @pl.when(i == 0)
        def _init():
            st_s[...] = st

        @pl.when(i != 0)
        def _acc():
            st_s[...] = st_s[...] + st

    @pl.when(i >= _GRID)
    def _phase_b():
        j = i - _GRID
        mu = st_s[0] / float(N)
        var = st_s[1] / float(N) - mu * mu
        scale = g_ref[...] * lax.rsqrt(var + EPS)
        shift = be_ref[...] - mu * scale
        h = a1_s[pl.ds(j * _BR, _BR), :] * scale[None, :] + shift[None, :]
        xw2 = jnp.dot(h, w2_ref[...],
                      preferred_element_type=jnp.float32, precision=_HIGH)
        xz2_ref[...] = xw2 * dis_ref[...]


def _lh_body(part_ref, xz_ref, dis_ref, b_ref, batch_ref,
             g2_ref, be2_ref, wf1_ref, bf1_ref, g3_ref, be3_ref,
             wf2_ref, bf2_ref, wf3_ref, bf3_ref,
             out_ref, st_s, q_s, cnt_s):
    i = pl.program_id(0)

    @pl.when(i < _GRID)
    def _phase_pool():
        acc = part_ref[0] + part_ref[1] + xz_ref[...]
        pre = acc * dis_ref[...] + b_ref[...][None, :]
        a = jnp.maximum(pre, 0.0)
        st = jnp.stack([jnp.sum(a, axis=0), jnp.sum(a * a, axis=0)], axis=0)
        gids = lax.broadcasted_iota(jnp.int32, (_BR, G), 1)
        oh = (batch_ref[...] == gids).astype(jnp.float32)
        q = lax.dot_general(oh, a, (((0,), (0,)), ((), ())),
                            preferred_element_type=jnp.float32,
                            precision=_HIGH)
        cnt = jnp.sum(oh, axis=0)[None, :]

        @pl.when(i == 0)
        def _init():
            st_s[...] = st
            q_s[...] = q
            cnt_s[...] = cnt

        @pl.when(i != 0)
        def _acc():
            st_s[...] = st_s[...] + st
            q_s[...] = q_s[...] + q
            cnt_s[...] = cnt_s[...] + cnt

    @pl.when(i == _GRID)
    def _phase_head():
        mu = st_s[0] / float(N)
        var = st_s[1] / float(N) - mu * mu
        scale = g2_ref[...] * lax.rsqrt(var + EPS)
        shift = be2_ref[...] - mu * scale
        p = q_s[...] * scale[None, :] + cnt_s[0][:, None] * shift[None, :]
        p = jnp.maximum(jnp.dot(p, wf1_ref[...],
                                preferred_element_type=jnp.float32,
                                precision=_HIGH) + bf1_ref[...][None, :], 0.0)
        mu3 = jnp.mean(p, axis=0)
        var3 = jnp.mean((p - mu3[None, :]) ** 2, axis=0)
        p = g3_ref[...][None, :] * (p - mu3[None, :]) * lax.rsqrt(var3 + EPS) \
            + be3_ref[...][None, :]
        p = jnp.maximum(jnp.dot(p, wf2_ref[...],
                                preferred_element_type=jnp.float32,
                                precision=_HIGH) + bf2_ref[...][None, :], 0.0)
        z = jnp.dot(p, wf3_ref[...],
                    preferred_element_type=jnp.float32,
                    precision=_HIGH) + bf3_ref[...][None, :]
        m = jnp.max(z, axis=1, keepdims=True)
        out_ref[...] = z - m - jnp.log(jnp.sum(jnp.exp(z - m), axis=1,
                                               keepdims=True))


def _full(shape):
    return pl.BlockSpec(shape, lambda i: tuple(0 for _ in shape))


def _rows(bshape):
    return pl.BlockSpec(bshape, lambda i: (i,) + tuple(0 for _ in bshape[1:]))


def _tc_mm1(x, degt, w1):
    return pl.pallas_call(
        _mm1_body,
        grid=(_GRID,),
        in_specs=[
            _rows((_BR, D)),
            pl.BlockSpec((NC, _BR, LANES), lambda i: (0, i, 0)),
            _full((D, 128)),
        ],
        out_specs=[_rows((_BR, 128)), _rows((_BR, 1))],
        out_shape=[jax.ShapeDtypeStruct((N, 128), jnp.float32),
                   jax.ShapeDtypeStruct((N, 1), jnp.float32)],
    )(x, degt, w1)


def _frz(bshape):
    return pl.BlockSpec(
        bshape,
        lambda i: (jnp.where(i < _GRID, i, _GRID - 1),)
        + tuple(0 for _ in bshape[1:]))


def _frz3(bshape):
    return pl.BlockSpec(
        bshape,
        lambda i: (0, jnp.where(i < _GRID, i, _GRID - 1))
        + tuple(0 for _ in bshape[2:]))


def _tc_lm1(part, xz, dis, b1, g1, be1, w2):
    return pl.pallas_call(
        _lm1_body,
        grid=(2 * _GRID,),
        in_specs=[
            _frz3((NC, _BR, 128)),
            _frz((_BR, 128)),
            pl.BlockSpec((_BR, 1), lambda i: (i % _GRID, 0)),
            _full((128,)),
            _full((128,)),
            _full((128,)),
            _full((128, 64)),
        ],
        out_specs=pl.BlockSpec((_BR, 64), lambda i: (i % _GRID, 0)),
        out_shape=jax.ShapeDtypeStruct((N, 64), jnp.float32),
        scratch_shapes=[pltpu.VMEM((N, 128), jnp.float32),
                        pltpu.VMEM((2, 128), jnp.float32)],
    )(part, xz, dis, b1, g1, be1, w2)


def _tc_lh(part, xz2, dis, b2, batch,
           g2, be2, wf1, bf1, g3, be3, wf2, bf2, wf3, bf3):
    return pl.pallas_call(
        _lh_body,
        grid=(_GRID + 1,),
        in_specs=[
            _frz3((NC, _BR, 64)),
            _frz((_BR, 64)),
            _frz((_BR, 1)),
            _full((64,)),
            _frz((_BR, 1)),
            _full((64,)), _full((64,)),
            _full((64, 64)), _full((64,)), _full((64,)), _full((64,)),
            _full((64, 64)), _full((64,)), _full((64, 2)), _full((2,)),
        ],
        out_specs=pl.BlockSpec((G, 2), lambda i: (0, 0)),
        out_shape=jax.ShapeDtypeStruct((G, 2), jnp.float32),
        scratch_shapes=[pltpu.VMEM((2, 64), jnp.float32),
                        pltpu.VMEM((G, 64), jnp.float32),
                        pltpu.VMEM((1, 64), jnp.float32)],
    )(part, xz2, dis, b2, batch,
      g2, be2, wf1, bf1, g3, be3, wf2, bf2, wf3, bf3)


def kernel(x, edge_index, batch, W1, b1, g1, be1, W2, b2, g2, be2,
           Wf1, bf1, g3, be3, Wf2, bf2, Wf3, bf3):
    pad_src = jnp.broadcast_to(jnp.arange(PADPT, dtype=jnp.int32) * 64,
                               (NW, PADPT))
    pad_dst = (N + (jnp.arange(NW, dtype=jnp.int32) // NC)[:, None] * DUMP
               + (jnp.arange(PADPT, dtype=jnp.int32) % DUMP)[None, :])
    srcf = jnp.concatenate([edge_index[0].reshape(NW, EPT), pad_src], axis=1)
    dstf = jnp.concatenate([edge_index[1].reshape(NW, EPT), pad_dst], axis=1)
    srcp = srcf.reshape(NW * NCHP, CHP)
    dstp = dstf.reshape(NW * NCHP, CHP)
    degt = _make_deg()(dstf.reshape(NW * NCH, CH))
    xz1, dis = _tc_mm1(x, degt, W1)
    part1 = _make_prop(128)(xz1, srcp, dstp)
    xz2 = _tc_lm1(part1, xz1, dis, b1, g1, be1, W2)
    part2 = _make_prop(64)(xz2, srcp, dstp)
    return _tc_lh(part2, xz2, dis, b2, batch[:, None],
                  g2, be2, Wf1, bf1, g3, be3, Wf2, bf2, Wf3, bf3)
